# Initial kernel scaffold; baseline (speedup 1.0000x reference)
import jax
import jax.numpy as jnp
from jax.experimental import pallas as pl


def kernel(batch, table):
    raise NotImplementedError("write your pallas kernel here")


# speedup vs baseline: 4.0862x; 4.0862x over previous
import functools

import jax
import jax.numpy as jnp
from jax import lax
from jax.experimental import pallas as pl
from jax.experimental.pallas import tpu as pltpu
from jax.experimental.pallas import tpu_sc as plsc

_BATCH = 4096
_SENT = 50
_DIM = 64

_NC = 2
_NS = 16
_NW = _NC * _NS
_B = _BATCH * _SENT
_BPW = _B // _NW
_CHUNK = 128
_NCHUNK = _BPW // _CHUNK


def _sc_embed(table, idx3):
    mesh = plsc.VectorSubcoreMesh(core_axis_name="c", subcore_axis_name="s")

    @functools.partial(
        pl.kernel,
        mesh=mesh,
        out_type=jax.ShapeDtypeStruct((_B, _DIM), jnp.float32),
        compiler_params=pltpu.CompilerParams(use_tc_tiling_on_sc=False),
        scratch_types=[
            pltpu.VMEM((_NCHUNK, _CHUNK), jnp.int32),
            pltpu.VMEM((_CHUNK, _DIM), jnp.float32),
            pltpu.SemaphoreType.DMA,
        ],
    )
    def body(table_hbm, idx_hbm, out_hbm, idx_v, rows, sem):
        wid = lax.axis_index("s") * _NC + lax.axis_index("c")
        base = wid * _BPW
        pltpu.sync_copy(idx_hbm.at[wid], idx_v)

        def step(j, carry):
            pltpu.async_copy(table_hbm.at[idx_v.at[j]], rows, sem).wait()
            pltpu.sync_copy(rows, out_hbm.at[pl.ds(base + j * _CHUNK, _CHUNK)])
            return carry

        lax.fori_loop(0, _NCHUNK, step, 0)

    return body(table, idx3)


def kernel(batch, table):
    idx3 = batch.reshape(_NW, _NCHUNK, _CHUNK)
    out = _sc_embed(table, idx3)
    return out.reshape(_BATCH, _SENT, _DIM)


# speedup vs baseline: 4.6436x; 1.1364x over previous
import functools

import jax
import jax.numpy as jnp
from jax import lax
from jax.experimental import pallas as pl
from jax.experimental.pallas import tpu as pltpu
from jax.experimental.pallas import tpu_sc as plsc

_BATCH = 4096
_SENT = 50
_DIM = 64

_NC = 2
_NS = 16
_NW = _NC * _NS
_B = _BATCH * _SENT
_BPW = _B // _NW
_CHUNK = 128
_NCHUNK = _BPW // _CHUNK
_K = 5
_SUP = _K * _CHUNK
_NSUP = _BPW // _SUP


def _sc_embed(table, idx3):
    mesh = plsc.VectorSubcoreMesh(core_axis_name="c", subcore_axis_name="s")

    @functools.partial(
        pl.kernel,
        mesh=mesh,
        out_type=jax.ShapeDtypeStruct((_B, _DIM), jnp.float32),
        compiler_params=pltpu.CompilerParams(use_tc_tiling_on_sc=False),
        scratch_types=[
            pltpu.VMEM((_NCHUNK, _CHUNK), jnp.int32),
            pltpu.VMEM((_SUP, _DIM), jnp.float32),
            pltpu.VMEM((_SUP, _DIM), jnp.float32),
            pltpu.SemaphoreType.DMA,
            pltpu.SemaphoreType.DMA,
        ],
    )
    def body(table_hbm, idx_hbm, out_hbm, idx_v, rows_a, rows_b, gs_a, gs_b):
        wid = lax.axis_index("s") * _NC + lax.axis_index("c")
        base = wid * _BPW
        pltpu.sync_copy(idx_hbm.at[wid], idx_v)

        def fire(s, rows, gs):
            for t in range(_K):
                pltpu.async_copy(
                    table_hbm.at[idx_v.at[s * _K + t]],
                    rows.at[pl.ds(t * _CHUNK, _CHUNK)],
                    gs,
                )

        def drain(rows, gs):
            pltpu.make_async_copy(out_hbm.at[pl.ds(0, _SUP)], rows, gs).wait()

        def flush(s, rows, gs):
            drain(rows, gs)
            pltpu.sync_copy(rows, out_hbm.at[pl.ds(base + s * _SUP, _SUP)])

        fire(0, rows_a, gs_a)
        fire(1, rows_b, gs_b)

        def step(i, carry):
            s = 2 * i
            flush(s - 2, rows_a, gs_a)
            fire(s, rows_a, gs_a)
            flush(s - 1, rows_b, gs_b)
            fire(s + 1, rows_b, gs_b)
            return carry

        lax.fori_loop(1, _NSUP // 2, step, 0)
        flush(_NSUP - 2, rows_a, gs_a)
        flush(_NSUP - 1, rows_b, gs_b)

    return body(table, idx3)


def kernel(batch, table):
    idx3 = batch.reshape(_NW, _NCHUNK, _CHUNK)
    out = _sc_embed(table, idx3)
    return out.reshape(_BATCH, _SENT, _DIM)
